# baseline (speedup 1.0000x reference)
import functools

import jax
import jax.numpy as jnp
from jax import lax
from jax.experimental import pallas as pl
from jax.experimental.pallas import tpu as pltpu
from jax.experimental.pallas import tpu_sc as plsc

_N = 10000
_D = 128
_E = 320000
_L = 3

_NC = 2
_NS = 16
_NW = _NC * _NS
_CH = 40
_EPT = _E // _NW
_NCH = _EPT // _CH
_NB = 4
_NPAD = _NS * 640
_ROWS_PT = _NPAD // _NS

_mesh = plsc.VectorSubcoreMesh(core_axis_name="c", subcore_axis_name="s")


@functools.partial(
    pl.kernel,
    out_type=jax.ShapeDtypeStruct((_NC, _NPAD, _D), jnp.float32),
    mesh=_mesh,
    scratch_types=(
        [pltpu.VMEM((2, _CH), jnp.int32)] * (2 * _NB)
        + [pltpu.VMEM((_CH, _D), jnp.float32)] * _NB
        + [pltpu.VMEM_SHARED((_NPAD, _D), jnp.float32)]
        + [pltpu.SemaphoreType.DMA] * (4 * _NB)
    ),
)
def _segsum_sc(h_hbm, idx_hbm, out_hbm, *refs):
    iba = list(refs[0:_NB])
    ibb = list(refs[_NB:2 * _NB])
    rbs = list(refs[2 * _NB:3 * _NB])
    acc = refs[3 * _NB]
    sgs = list(refs[3 * _NB + 1:4 * _NB + 1])
    sss = list(refs[4 * _NB + 1:5 * _NB + 1])
    sia = list(refs[5 * _NB + 1:6 * _NB + 1])
    sib = list(refs[6 * _NB + 1:7 * _NB + 1])
    rb0 = rbs[0]
    c = lax.axis_index("c")
    s = lax.axis_index("s")
    w = c * _NS + s

    zvec = jnp.zeros((16,), jnp.float32)

    def _zstore(i, carry):
        rb0[i // (_D // 16), pl.ds((i % (_D // 16)) * 16, 16)] = zvec
        return carry

    lax.fori_loop(0, _CH * (_D // 16), _zstore, 0)

    def _zcopy(i, carry):
        pltpu.sync_copy(rb0, acc.at[pl.ds(s * 640 + i * _CH, _CH)])
        return carry

    lax.fori_loop(0, 640 // _CH, _zcopy, 0)
    plsc.subcore_barrier()

    def _sets(k):
        return (iba, sia) if (k // _NB) % 2 == 0 else (ibb, sib)

    def _pos(c_dyn, k, guard6):
        b, f = k % _NB, (k + 2) % _NB
        cur, _ = _sets(k)
        set2, si2 = _sets(k + 2)
        set6, si6 = _sets(k + 6)
        pltpu.make_async_copy(h_hbm.at[cur[b].at[0]], rbs[b], sgs[b]).wait()
        pltpu.async_copy(rbs[b], acc.at[cur[b].at[1]], sss[b], add=True)
        pltpu.make_async_copy(rbs[f], acc.at[cur[f].at[1]], sss[f]).wait()
        if guard6:
            @pl.when(c_dyn + 6 < _NCH)
            def _():
                pltpu.async_copy(idx_hbm.at[w, c_dyn + 6], set6[f], si6[f])
        else:
            pltpu.async_copy(idx_hbm.at[w, c_dyn + 6], set6[f], si6[f])
        pltpu.make_async_copy(idx_hbm.at[w, c_dyn + 2], set2[f], si2[f]).wait()
        pltpu.async_copy(h_hbm.at[set2[f].at[0]], rbs[f], sgs[f])

    for b in range(_NB):
        pltpu.sync_copy(idx_hbm.at[w, b], iba[b])
        pltpu.async_copy(idx_hbm.at[w, _NB + b], ibb[b], sib[b])
        pltpu.async_copy(h_hbm.at[iba[b].at[0]], rbs[b], sgs[b])

    for k in range(2):
        pltpu.make_async_copy(h_hbm.at[iba[k].at[0]], rbs[k], sgs[k]).wait()
        pltpu.async_copy(rbs[k], acc.at[iba[k].at[1]], sss[k], add=True)
    for k in range(2, 2 * _NB):
        _pos(k, k, guard6=False)

    def _body(i, carry):
        c0 = 2 * _NB * i
        for k in range(2 * _NB):
            _pos(c0 + k, k, guard6=True)
        return carry

    lax.fori_loop(1, _NCH // (2 * _NB), _body, 0)

    for t in range((_NCH // (2 * _NB)) * 2 * _NB, _NCH):
        k = t % (2 * _NB)
        b = k % _NB
        cur, _ = _sets(k)
        pltpu.make_async_copy(h_hbm.at[cur[b].at[0]], rbs[b], sgs[b]).wait()
        pltpu.async_copy(rbs[b], acc.at[cur[b].at[1]], sss[b], add=True)
    for b in range(_NB):
        pltpu.make_async_copy(rbs[b], acc.at[iba[b].at[1]], sss[b]).wait()
    plsc.subcore_barrier()
    pltpu.sync_copy(acc.at[pl.ds(s * _ROWS_PT, _ROWS_PT)],
                    out_hbm.at[c, pl.ds(s * _ROWS_PT, _ROWS_PT)])


def _bn(z, gamma, beta, relu):
    mean = jnp.mean(z, axis=0, keepdims=True)
    zc = z - mean
    var = jnp.mean(zc * zc, axis=0, keepdims=True)
    out = gamma * zc * lax.rsqrt(var + 1e-5) + beta
    return jnp.maximum(out, 0.0) if relu else out


def _tc_layer_body(h_ref, p_ref, w1t_ref, w2t_ref, g1_ref, b1_ref, ga_ref,
                   ba_ref, go_ref, bo_ref, out_ref, *, relu_out):
    x = h_ref[...] + p_ref[0, :_N] + p_ref[1, :_N]
    z = jnp.dot(x, w1t_ref[...], preferred_element_type=jnp.float32)
    z = _bn(z, g1_ref[...], b1_ref[...], relu=True)
    z = jnp.dot(z, w2t_ref[...], preferred_element_type=jnp.float32)
    z = _bn(z, ga_ref[...], ba_ref[...], relu=True)
    out_ref[...] = _bn(z, go_ref[...], bo_ref[...], relu=relu_out)


def _tc_layer(relu_out):
    return pl.pallas_call(
        functools.partial(_tc_layer_body, relu_out=relu_out),
        out_shape=jax.ShapeDtypeStruct((_N, _D), jnp.float32),
    )


def kernel(h, edge_index, W1, W2, mlp_bn_gamma, mlp_bn_beta, apply_bn_gamma,
           apply_bn_beta, out_bn_gamma, out_bn_beta):
    idx = jnp.stack([edge_index[0].reshape(_NW, _NCH, _CH),
                     edge_index[1].reshape(_NW, _NCH, _CH)], axis=2)
    for i in range(_L):
        parts = _segsum_sc(h, idx)
        h = _tc_layer(i != _L - 1)(
            h, parts,
            W1[i].T, W2[i].T,
            mlp_bn_gamma[i].reshape(1, _D), mlp_bn_beta[i].reshape(1, _D),
            apply_bn_gamma[i].reshape(1, _D), apply_bn_beta[i].reshape(1, _D),
            out_bn_gamma[i].reshape(1, _D), out_bn_beta[i].reshape(1, _D),
        )
    return h


# speedup vs baseline: 1.3742x; 1.3742x over previous
import functools

import jax
import jax.numpy as jnp
from jax import lax
from jax.experimental import pallas as pl
from jax.experimental.pallas import tpu as pltpu
from jax.experimental.pallas import tpu_sc as plsc

_N = 10000
_D = 128
_E = 320000
_L = 3

_NC = 2
_NS = 16
_NW = _NC * _NS
_CH = 80
_EPT = _E // _NW
_NCH = _EPT // _CH
_NB = 3
_NMAIN = (_NCH // (2 * _NB)) * 2 * _NB
_NPAD = _NS * 640
_ROWS_PT = _NPAD // _NS

_mesh = plsc.VectorSubcoreMesh(core_axis_name="c", subcore_axis_name="s")


@functools.partial(
    pl.kernel,
    out_type=jax.ShapeDtypeStruct((_NC, _NPAD, _D), jnp.float32),
    mesh=_mesh,
    scratch_types=(
        [pltpu.VMEM((2, _CH), jnp.int32)] * (2 * _NB)
        + [pltpu.VMEM((_CH, _D), jnp.float32)] * _NB
        + [pltpu.VMEM_SHARED((_NPAD, _D), jnp.float32)]
        + [pltpu.SemaphoreType.DMA] * (3 * _NB)
    ),
)
def _segsum_sc(h_hbm, idx_hbm, out_hbm, *refs):
    iba = list(refs[0:_NB])
    ibb = list(refs[_NB:2 * _NB])
    rbs = list(refs[2 * _NB:3 * _NB])
    acc = refs[3 * _NB]
    sgs = list(refs[3 * _NB + 1:4 * _NB + 1])
    sia = list(refs[4 * _NB + 1:5 * _NB + 1])
    sib = list(refs[5 * _NB + 1:6 * _NB + 1])
    rb0 = rbs[0]
    c = lax.axis_index("c")
    s = lax.axis_index("s")
    w = c * _NS + s

    zvec = jnp.zeros((16,), jnp.float32)

    def _zstore(i, carry):
        rb0[i // (_D // 16), pl.ds((i % (_D // 16)) * 16, 16)] = zvec
        return carry

    lax.fori_loop(0, _CH * (_D // 16), _zstore, 0)

    def _zcopy(i, carry):
        pltpu.sync_copy(rb0, acc.at[pl.ds(s * 640 + i * _CH, _CH)])
        return carry

    lax.fori_loop(0, 640 // _CH, _zcopy, 0)
    plsc.subcore_barrier()

    for b in range(_NB):
        pltpu.sync_copy(idx_hbm.at[w, b], iba[b])
        pltpu.async_copy(idx_hbm.at[w, _NB + b], ibb[b], sib[b])
        pltpu.async_copy(h_hbm.at[iba[b].at[0]], rbs[b], sgs[b])

    def _body(i, carry):
        c0 = 2 * _NB * i
        for half in range(2):
            ibs, sis = (iba, sia) if half == 0 else (ibb, sib)
            ibo, sio = (ibb, sib) if half == 0 else (iba, sia)
            for b in range(_NB):
                ch = c0 + half * _NB + b
                pltpu.make_async_copy(
                    h_hbm.at[ibs[b].at[0]], rbs[b], sgs[b]).wait()
                pltpu.sync_copy(rbs[b], acc.at[ibs[b].at[1]], add=True)

                @pl.when(ch + 2 * _NB < _NCH)
                def _():
                    pltpu.async_copy(idx_hbm.at[w, ch + 2 * _NB], ibs[b],
                                     sis[b])

                pltpu.make_async_copy(idx_hbm.at[w, ch + _NB], ibo[b],
                                      sio[b]).wait()
                pltpu.async_copy(h_hbm.at[ibo[b].at[0]], rbs[b], sgs[b])

        return carry

    lax.fori_loop(0, _NMAIN // (2 * _NB), _body, 0)

    for t in range(_NMAIN, _NCH):
        b = t % _NB
        ibs = iba if (t // _NB) % 2 == 0 else ibb
        pltpu.make_async_copy(h_hbm.at[ibs[b].at[0]], rbs[b], sgs[b]).wait()
        pltpu.sync_copy(rbs[b], acc.at[ibs[b].at[1]], add=True)
        if t + _NB < _NCH:
            ibo = ibb if (t // _NB) % 2 == 0 else iba
            sio = sib if (t // _NB) % 2 == 0 else sia
            pltpu.make_async_copy(idx_hbm.at[w, t + _NB], ibo[b],
                                  sio[b]).wait()
            pltpu.async_copy(h_hbm.at[ibo[b].at[0]], rbs[b], sgs[b])
    plsc.subcore_barrier()
    pltpu.sync_copy(acc.at[pl.ds(s * _ROWS_PT, _ROWS_PT)],
                    out_hbm.at[c, pl.ds(s * _ROWS_PT, _ROWS_PT)])


def _bn(z, gamma, beta, relu):
    mean = jnp.mean(z, axis=0, keepdims=True)
    zc = z - mean
    var = jnp.mean(zc * zc, axis=0, keepdims=True)
    out = gamma * zc * lax.rsqrt(var + 1e-5) + beta
    return jnp.maximum(out, 0.0) if relu else out


def _tc_layer_body(h_ref, p_ref, w1t_ref, w2t_ref, g1_ref, b1_ref, ga_ref,
                   ba_ref, go_ref, bo_ref, out_ref, *, relu_out):
    x = h_ref[...] + p_ref[0, :_N] + p_ref[1, :_N]
    z = jnp.dot(x, w1t_ref[...], preferred_element_type=jnp.float32)
    z = _bn(z, g1_ref[...], b1_ref[...], relu=True)
    z = jnp.dot(z, w2t_ref[...], preferred_element_type=jnp.float32)
    z = _bn(z, ga_ref[...], ba_ref[...], relu=True)
    out_ref[...] = _bn(z, go_ref[...], bo_ref[...], relu=relu_out)


def _tc_layer(relu_out):
    return pl.pallas_call(
        functools.partial(_tc_layer_body, relu_out=relu_out),
        out_shape=jax.ShapeDtypeStruct((_N, _D), jnp.float32),
    )


def kernel(h, edge_index, W1, W2, mlp_bn_gamma, mlp_bn_beta, apply_bn_gamma,
           apply_bn_beta, out_bn_gamma, out_bn_beta):
    idx = jnp.stack([edge_index[0].reshape(_NW, _NCH, _CH),
                     edge_index[1].reshape(_NW, _NCH, _CH)], axis=2)
    for i in range(_L):
        parts = _segsum_sc(h, idx)
        h = _tc_layer(i != _L - 1)(
            h, parts,
            W1[i].T, W2[i].T,
            mlp_bn_gamma[i].reshape(1, _D), mlp_bn_beta[i].reshape(1, _D),
            apply_bn_gamma[i].reshape(1, _D), apply_bn_beta[i].reshape(1, _D),
            out_bn_gamma[i].reshape(1, _D), out_bn_beta[i].reshape(1, _D),
        )
    return h
